# baseline (speedup 1.0000x reference)
import functools

import jax
import jax.numpy as jnp
from jax import lax
from jax.experimental import pallas as pl
from jax.experimental.pallas import tpu as pltpu
from jax.experimental.pallas import tpu_sc as plsc

D = 256
K = 8192
N = 8192
BETA = 0.25
TILE = 256
STEPS = N // TILE


_ONE_BITS = 0x3F800000


def _vq_kernel(x_ref, esq_ref, emb_ref,
               dist_ref, enc_ref, ids_ref, loss_ref, perp_ref, embt_ref,
               counts_ref, msum_ref):
    i = pl.program_id(0)

    @pl.when(i == 0)
    def _embt():
        embt_ref[...] = jnp.transpose(emb_ref[...])
    xt = x_ref[0]
    x2t = -2.0 * xt
    xsq = jnp.transpose(jnp.sum(xt * xt, axis=0, keepdims=True))
    cross2 = jax.lax.dot_general(x2t, emb_ref[...],
                                 (((0,), (0,)), ((), ())),
                                 preferred_element_type=jnp.float32)
    dist = (xsq + esq_ref[...]) + cross2
    dist_ref[...] = dist

    m = jnp.min(dist, axis=1, keepdims=True)
    col = jax.lax.broadcasted_iota(jnp.int32, (TILE, K), 1)
    colb = col + _ONE_BITS
    colb_f = lax.bitcast_convert_type(colb, jnp.float32)
    idb_f = jnp.min(jnp.where(dist == m, colb_f, 3.0), axis=1)
    idb = lax.bitcast_convert_type(idb_f, jnp.int32)
    enc = (colb == idb[:, None]).astype(jnp.float32)
    enc_ref[...] = enc
    ids_ref[...] = (idb - _ONE_BITS).reshape(1, 1, TILE)

    part_counts = jnp.sum(enc, axis=0, keepdims=True)
    part_msum = jnp.sum(m)

    @pl.when(i == 0)
    def _init():
        counts_ref[...] = part_counts
        msum_ref[0, 0] = part_msum

    @pl.when(i > 0)
    def _acc():
        counts_ref[...] += part_counts
        msum_ref[0, 0] += part_msum

    @pl.when(i == STEPS - 1)
    def _fin():
        loss_val = BETA * msum_ref[0, 0] / (N * D)
        loss_ref[...] = jnp.full((1, 1), loss_val, jnp.float32)
        p = counts_ref[...] * (1.0 / N)
        ent = jnp.sum(p * jnp.log(p + 1e-10))
        perp_ref[...] = jnp.full((1, 1), jnp.exp(-ent), jnp.float32)


@functools.cache
def _sc_gather_fn():
    info = plsc.get_sparse_core_info()
    nc = info.num_cores
    bpw = N // (nc * info.num_subcores)

    @functools.partial(
        pl.kernel,
        mesh=plsc.VectorSubcoreMesh(core_axis_name="c", subcore_axis_name="s"),
        out_type=jax.ShapeDtypeStruct((N, D), jnp.float32),
        scratch_types=[
            pltpu.VMEM((bpw,), jnp.int32),
            pltpu.VMEM((bpw, D), jnp.float32),
            pltpu.SemaphoreType.DMA,
        ],
    )
    def _sc_gather(table_hbm, idx_hbm, out_hbm, idx_v, rows_v, sem):
        wid = lax.axis_index("s") * nc + lax.axis_index("c")
        base = wid * bpw
        pltpu.sync_copy(idx_hbm.at[pl.ds(base, bpw)], idx_v)
        pltpu.async_copy(table_hbm.at[idx_v], rows_v, sem).wait()
        pltpu.sync_copy(rows_v, out_hbm.at[pl.ds(base, bpw)])

    return _sc_gather


def kernel(x, embedding):
    B, _, H, W = x.shape
    x3 = x.reshape(B, D, H * W)
    esq = jnp.sum(embedding ** 2, axis=0, keepdims=True)
    tiles_per_b = (H * W) // TILE

    dist_out, enc_out, ids_out, loss_out, perp_out, embt_out = pl.pallas_call(
        _vq_kernel,
        grid=(STEPS,),
        in_specs=[
            pl.BlockSpec((1, D, TILE),
                         lambda i: (i // tiles_per_b, 0, i % tiles_per_b)),
            pl.BlockSpec((1, K), lambda i: (0, 0)),
            pl.BlockSpec((D, K), lambda i: (0, 0)),
        ],
        out_specs=[
            pl.BlockSpec((TILE, K), lambda i: (i, 0)),
            pl.BlockSpec((TILE, K), lambda i: (i, 0)),
            pl.BlockSpec((1, 1, TILE), lambda i: (i, 0, 0)),
            pl.BlockSpec((1, 1), lambda i: (0, 0)),
            pl.BlockSpec((1, 1), lambda i: (0, 0)),
            pl.BlockSpec((K, D), lambda i: (0, 0)),
        ],
        out_shape=[
            jax.ShapeDtypeStruct((N, K), jnp.float32),
            jax.ShapeDtypeStruct((N, K), jnp.float32),
            jax.ShapeDtypeStruct((STEPS, 1, TILE), jnp.int32),
            jax.ShapeDtypeStruct((1, 1), jnp.float32),
            jax.ShapeDtypeStruct((1, 1), jnp.float32),
            jax.ShapeDtypeStruct((K, D), jnp.float32),
        ],
        scratch_shapes=[
            pltpu.VMEM((1, K), jnp.float32),
            pltpu.SMEM((1, 1), jnp.float32),
        ],
        compiler_params=pltpu.CompilerParams(
            dimension_semantics=("arbitrary",),
        ),
    )(x3, esq, embedding)

    ids_flat = ids_out.reshape(N)
    quant_flat = _sc_gather_fn()(embt_out, ids_flat)

    out = jnp.transpose(quant_flat.reshape(B, H, W, D), (0, 3, 1, 2))
    loss = loss_out[0, 0]
    perplexity = perp_out[0, 0]
    ids_grid = ids_out.reshape(B, H, W)
    return (out, loss, perplexity, enc_out, ids_grid, dist_out)
